# baseline (speedup 1.0000x reference)
import functools

import jax
import jax.numpy as jnp
from jax import lax
from jax.experimental import pallas as pl
from jax.experimental.pallas import tpu as pltpu
from jax.experimental.pallas import tpu_sc as plsc

NC = 2
NS = 16
L = 16
NW = NC * NS
NBUF = 4
CH = 5


def _project(emb_table, fc_w, scale):
    V, E = emb_table.shape
    O = fc_w.shape[0]
    emb_t = jnp.transpose(emb_table)
    BLKC = 4096

    def body(w_ref, emb_ref, out_ref):
        out_ref[...] = lax.dot_general(
            w_ref[...], emb_ref[...],
            (((1,), (0,)), ((), ())),
            preferred_element_type=jnp.float32,
        ) * scale

    pt = pl.pallas_call(
        body,
        grid=(pl.cdiv(V, BLKC),),
        in_specs=[
            pl.BlockSpec((O, E), lambda i: (0, 0)),
            pl.BlockSpec((E, BLKC), lambda i: (0, i)),
        ],
        out_specs=pl.BlockSpec((O, BLKC), lambda i: (0, i)),
        out_shape=jax.ShapeDtypeStruct((O, V), jnp.float32),
    )(fc_w, emb_t)
    PK = 128 // O
    p = pt.reshape(O, V // PK, PK).transpose(1, 2, 0).reshape(V // PK, PK * O)
    return p.reshape(V, O)


def _make_sc_pool(S, B, D):
    BW = B // NW
    assert B % NW == 0 and BW % L == 0 and D == L

    mesh = plsc.VectorSubcoreMesh(core_axis_name="c", subcore_axis_name="s")

    @functools.partial(
        pl.kernel,
        out_type=jax.ShapeDtypeStruct((B, D), jnp.float32),
        mesh=mesh,
        compiler_params=pltpu.CompilerParams(use_tc_tiling_on_sc=False),
        scratch_types=[
            pltpu.VMEM((S // CH, CH * BW), jnp.int32),
            pltpu.VMEM((2 * NBUF, CH * BW, D), jnp.float32),
            pltpu.VMEM((BW, D), jnp.float32),
            pltpu.VMEM((CH * BW,), jnp.int32),
            pltpu.VMEM((L,), jnp.float32),
            pltpu.VMEM_SHARED((CH * NS * BW, D), jnp.float32),
            pltpu.SemaphoreType.DMA((2 * NBUF,)),
            pltpu.SemaphoreType.DMA((2 * NBUF,)),
            pltpu.SemaphoreType.DMA,
        ],
    )
    def sc_pool(text_hbm, p_hbm, b_hbm, out_hbm,
                idx_v, rows_v, stage_v, scat_v, bias_v, acc_sh, gsems, ssems,
                stage_sem):
        cid = lax.axis_index("c")
        sid = lax.axis_index("s")
        wid = sid * NC + cid
        base = wid * BW
        sbase = sid * BW
        NSTEP = S // CH
        NSLOT = 2 * NBUF
        ROUNDS = NSTEP // NSLOT
        assert S % CH == 0 and NSTEP % NSLOT == 0

        def stage_row(r, _):
            s = r // CH
            c = r % CH
            pltpu.async_copy(text_hbm.at[pl.ds(r * B + base, BW)],
                             idx_v.at[s, pl.ds(c * BW, BW)], stage_sem)
            return 0
        lax.fori_loop(0, S, stage_row, 0)
        pltpu.sync_copy(b_hbm, bias_v)
        bias = bias_v[...]

        def stage_drain(r, _):
            pltpu.make_async_copy(
                text_hbm.at[pl.ds(r * B + base, BW)],
                idx_v.at[r // CH, pl.ds((r % CH) * BW, BW)], stage_sem).wait()
            return 0
        lax.fori_loop(0, S, stage_drain, 0)

        def init_row(j, _):
            stage_v[j] = bias
            return 0
        lax.fori_loop(0, BW, init_row, 0)
        pltpu.sync_copy(stage_v, acc_sh.at[pl.ds(sbase, BW)])

        def zero_row(j, _):
            stage_v[j] = jnp.zeros((L,), jnp.float32)
            return 0
        lax.fori_loop(0, BW, zero_row, 0)
        for c in range(1, CH):
            pltpu.sync_copy(stage_v, acc_sh.at[pl.ds(c * NS * BW + sbase, BW)])

        PERB = BW // L
        def init_scat(i, _):
            c = i // PERB
            scat_v[pl.ds(i * L, L)] = (
                lax.iota(jnp.int32, L)
                + (c * (NS * BW) + sbase + (i % PERB) * L))
            return 0
        lax.fori_loop(0, CH * BW // L, init_scat, 0)

        def gather_start(s, slot):
            pltpu.async_copy(p_hbm.at[idx_v.at[s]], rows_v.at[slot],
                             gsems.at[slot])

        def gather_wait(s, slot):
            pltpu.make_async_copy(
                p_hbm.at[idx_v.at[s]], rows_v.at[slot], gsems.at[slot]).wait()

        def scat_start(slot):
            pltpu.async_copy(rows_v.at[slot], acc_sh.at[scat_v],
                             ssems.at[slot], add=True)

        def scat_wait(slot):
            pltpu.make_async_copy(rows_v.at[slot], acc_sh.at[scat_v],
                                  ssems.at[slot]).wait()

        for b in range(NBUF):
            gather_start(b, b)

        for b in range(NSLOT):
            gather_wait(b, b)
            scat_start(b)
            if b + NBUF < NSLOT:
                gather_start(b + NBUF, (b + NBUF) % NSLOT)
            else:
                scat_wait((b + NBUF) % NSLOT)
                gather_start(b + NBUF, (b + NBUF) % NSLOT)

        def steady(k, _):
            s0 = k * NSLOT
            for b in range(NSLOT):
                s = s0 + b
                gather_wait(s, b)
                scat_start(b)
                slot2 = (b + NBUF) % NSLOT
                scat_wait(slot2)
                gather_start(s + NBUF, slot2)
            return 0
        lax.fori_loop(1, ROUNDS - 1, steady, 0)

        s0 = NSTEP - NSLOT
        for b in range(NSLOT):
            s = s0 + b
            gather_wait(s, b)
            scat_start(b)
            if s + NBUF < NSTEP:
                slot2 = (b + NBUF) % NSLOT
                scat_wait(slot2)
                gather_start(s + NBUF, slot2)

        for b in range(NSLOT):
            scat_wait(b)

        for c in range(CH):
            pltpu.sync_copy(acc_sh.at[pl.ds(c * NS * BW + sbase, BW)],
                            rows_v.at[0, pl.ds(c * BW, BW)])

        def sum_banks(j, _):
            acc = rows_v[0, j]
            for c in range(1, CH):
                acc = acc + rows_v[0, c * BW + j]
            stage_v[j] = acc
            return 0
        lax.fori_loop(0, BW, sum_banks, 0)
        pltpu.sync_copy(stage_v, out_hbm.at[pl.ds(base, BW)])

    return sc_pool


def kernel(text, emb_table, fc_w, fc_b):
    S, B = text.shape
    V = emb_table.shape[0]
    D = fc_w.shape[0]
    text1 = jnp.minimum(text.reshape(S * B), jnp.int32(V - 1))
    p = _project(emb_table, fc_w, 1.0 / S)
    return _make_sc_pool(S, B, D)(text1, p, fc_b)


# speedup vs baseline: 1.3681x; 1.3681x over previous
import functools

import jax
import jax.numpy as jnp
from jax import lax
from jax.experimental import pallas as pl
from jax.experimental.pallas import tpu as pltpu
from jax.experimental.pallas import tpu_sc as plsc

NC = 2
NS = 16
L = 16
NW = NC * NS
NBUF = 4
CH = 5


def _project(emb_table, fc_w, scale):
    V, E = emb_table.shape
    O = fc_w.shape[0]
    emb_t = jnp.transpose(emb_table)
    BLKC = 4096

    PK = 128 // O
    eye = jnp.eye(O, dtype=jnp.float32)

    def body(w_ref, emb_ref, eye_ref, out_ref):
        pt = lax.dot_general(
            w_ref[...], emb_ref[...],
            (((1,), (0,)), ((), ())),
            preferred_element_type=jnp.float32,
        ) * scale
        pb = lax.dot_general(
            pt, eye_ref[...],
            (((0,), (0,)), ((), ())),
            preferred_element_type=jnp.float32,
        )
        SUB = BLKC // PK
        for j in range(PK):
            out_ref[:, j * O:(j + 1) * O] = pb[j * SUB:(j + 1) * SUB, :]

    NBLK = pl.cdiv(V, BLKC)
    p_pk = pl.pallas_call(
        body,
        grid=(NBLK,),
        in_specs=[
            pl.BlockSpec((O, E), lambda i: (0, 0)),
            pl.BlockSpec((E, BLKC), lambda i: (0, i)),
            pl.BlockSpec((O, O), lambda i: (0, 0)),
        ],
        out_specs=pl.BlockSpec((BLKC // PK, PK * O), lambda i: (i, 0)),
        out_shape=jax.ShapeDtypeStruct((NBLK * BLKC // PK, PK * O), jnp.float32),
    )(fc_w, emb_t, eye)
    return p_pk.reshape(NBLK * BLKC, O)


def _make_sc_pool(S, B, D):
    BW = B // NW
    assert B % NW == 0 and BW % L == 0 and D == L

    mesh = plsc.VectorSubcoreMesh(core_axis_name="c", subcore_axis_name="s")

    @functools.partial(
        pl.kernel,
        out_type=jax.ShapeDtypeStruct((B, D), jnp.float32),
        mesh=mesh,
        compiler_params=pltpu.CompilerParams(use_tc_tiling_on_sc=False),
        scratch_types=[
            pltpu.VMEM((S // CH, CH * BW), jnp.int32),
            pltpu.VMEM((2 * NBUF, CH * BW, D), jnp.float32),
            pltpu.VMEM((BW, D), jnp.float32),
            pltpu.VMEM((CH * BW,), jnp.int32),
            pltpu.VMEM((L,), jnp.float32),
            pltpu.VMEM_SHARED((CH * NS * BW, D), jnp.float32),
            pltpu.SemaphoreType.DMA((2 * NBUF,)),
            pltpu.SemaphoreType.DMA((2 * NBUF,)),
            pltpu.SemaphoreType.DMA,
        ],
    )
    def sc_pool(text_hbm, p_hbm, b_hbm, out_hbm,
                idx_v, rows_v, stage_v, scat_v, bias_v, acc_sh, gsems, ssems,
                stage_sem):
        cid = lax.axis_index("c")
        sid = lax.axis_index("s")
        wid = sid * NC + cid
        base = wid * BW
        sbase = sid * BW
        NSTEP = S // CH
        NSLOT = 2 * NBUF
        ROUNDS = NSTEP // NSLOT
        assert S % CH == 0 and NSTEP % NSLOT == 0

        def stage_row(r, _):
            s = r // CH
            c = r % CH
            pltpu.async_copy(text_hbm.at[pl.ds(r * B + base, BW)],
                             idx_v.at[s, pl.ds(c * BW, BW)], stage_sem)
            return 0
        lax.fori_loop(0, S, stage_row, 0)
        pltpu.sync_copy(b_hbm, bias_v)
        bias = bias_v[...]

        def stage_drain(r, _):
            pltpu.make_async_copy(
                text_hbm.at[pl.ds(r * B + base, BW)],
                idx_v.at[r // CH, pl.ds((r % CH) * BW, BW)], stage_sem).wait()
            return 0
        lax.fori_loop(0, S, stage_drain, 0)

        def init_row(j, _):
            stage_v[j] = bias
            return 0
        lax.fori_loop(0, BW, init_row, 0)
        pltpu.sync_copy(stage_v, acc_sh.at[pl.ds(sbase, BW)])

        def zero_row(j, _):
            stage_v[j] = jnp.zeros((L,), jnp.float32)
            return 0
        lax.fori_loop(0, BW, zero_row, 0)
        for c in range(1, CH):
            pltpu.sync_copy(stage_v, acc_sh.at[pl.ds(c * NS * BW + sbase, BW)])

        PERB = BW // L
        def init_scat(i, _):
            c = i // PERB
            scat_v[pl.ds(i * L, L)] = (
                lax.iota(jnp.int32, L)
                + (c * (NS * BW) + sbase + (i % PERB) * L))
            return 0
        lax.fori_loop(0, CH * BW // L, init_scat, 0)

        def gather_start(s, slot):
            pltpu.async_copy(p_hbm.at[idx_v.at[s]], rows_v.at[slot],
                             gsems.at[slot])

        def gather_wait(s, slot):
            pltpu.make_async_copy(
                p_hbm.at[idx_v.at[s]], rows_v.at[slot], gsems.at[slot]).wait()

        def scat_start(slot):
            pltpu.async_copy(rows_v.at[slot], acc_sh.at[scat_v],
                             ssems.at[slot], add=True)

        def scat_wait(slot):
            pltpu.make_async_copy(rows_v.at[slot], acc_sh.at[scat_v],
                                  ssems.at[slot]).wait()

        for b in range(NBUF):
            gather_start(b, b)

        for b in range(NSLOT):
            gather_wait(b, b)
            scat_start(b)
            if b + NBUF < NSLOT:
                gather_start(b + NBUF, (b + NBUF) % NSLOT)
            else:
                scat_wait((b + NBUF) % NSLOT)
                gather_start(b + NBUF, (b + NBUF) % NSLOT)

        def steady(k, _):
            s0 = k * NSLOT
            for b in range(NSLOT):
                s = s0 + b
                gather_wait(s, b)
                scat_start(b)
                slot2 = (b + NBUF) % NSLOT
                scat_wait(slot2)
                gather_start(s + NBUF, slot2)
            return 0
        lax.fori_loop(1, ROUNDS - 1, steady, 0)

        s0 = NSTEP - NSLOT
        for b in range(NSLOT):
            s = s0 + b
            gather_wait(s, b)
            scat_start(b)
            if s + NBUF < NSTEP:
                slot2 = (b + NBUF) % NSLOT
                scat_wait(slot2)
                gather_start(s + NBUF, slot2)

        for b in range(NSLOT):
            scat_wait(b)

        for c in range(CH):
            pltpu.sync_copy(acc_sh.at[pl.ds(c * NS * BW + sbase, BW)],
                            rows_v.at[0, pl.ds(c * BW, BW)])

        def sum_banks(j, _):
            acc = rows_v[0, j]
            for c in range(1, CH):
                acc = acc + rows_v[0, c * BW + j]
            stage_v[j] = acc
            return 0
        lax.fori_loop(0, BW, sum_banks, 0)
        pltpu.sync_copy(stage_v, out_hbm.at[pl.ds(base, BW)])

    return sc_pool


def kernel(text, emb_table, fc_w, fc_b):
    S, B = text.shape
    V = emb_table.shape[0]
    D = fc_w.shape[0]
    BLKC, PK = 4096, 128 // D
    SUB = BLKC // PK
    g = jnp.minimum(text.reshape(S * B), jnp.int32(V - 1))
    gidx = ((g // BLKC) * SUB + (g % SUB)) * PK + (g % BLKC) // SUB
    p = _project(emb_table, fc_w, 1.0 / S)
    return _make_sc_pool(S, B, D)(gidx, p, fc_b)


# speedup vs baseline: 1.3939x; 1.0189x over previous
import functools

import jax
import jax.numpy as jnp
from jax import lax
from jax.experimental import pallas as pl
from jax.experimental.pallas import tpu as pltpu
from jax.experimental.pallas import tpu_sc as plsc

NC = 2
NS = 16
L = 16
NW = NC * NS
NBUF = 4
CH = 5


def _project(emb_table, fc_w, scale):
    V, E = emb_table.shape
    O = fc_w.shape[0]
    emb_t = jnp.transpose(emb_table)
    BLKC = 4096

    PK = 128 // O

    def body(w_ref, emb_ref, out_ref):
        SUB = BLKC // PK
        pieces = []
        for j in range(PK):
            pieces.append(lax.dot_general(
                emb_ref[:, j * SUB:(j + 1) * SUB], w_ref[...],
                (((0,), (1,)), ((), ())),
                preferred_element_type=jnp.float32,
            ) * scale)
        out_ref[...] = jnp.concatenate(pieces, axis=1)

    NBLK = pl.cdiv(V, BLKC)
    p_pk = pl.pallas_call(
        body,
        grid=(NBLK,),
        in_specs=[
            pl.BlockSpec((O, E), lambda i: (0, 0)),
            pl.BlockSpec((E, BLKC), lambda i: (0, i)),
        ],
        out_specs=pl.BlockSpec((BLKC // PK, PK * O), lambda i: (i, 0)),
        out_shape=jax.ShapeDtypeStruct((NBLK * BLKC // PK, PK * O), jnp.float32),
    )(fc_w, emb_t)
    return p_pk.reshape(NBLK * BLKC, O)


def _make_sc_pool(S, B, D):
    BW = B // NW
    assert B % NW == 0 and BW % L == 0 and D == L

    mesh = plsc.VectorSubcoreMesh(core_axis_name="c", subcore_axis_name="s")

    @functools.partial(
        pl.kernel,
        out_type=jax.ShapeDtypeStruct((B, D), jnp.float32),
        mesh=mesh,
        compiler_params=pltpu.CompilerParams(use_tc_tiling_on_sc=False),
        scratch_types=[
            pltpu.VMEM((S // CH, CH * BW), jnp.int32),
            pltpu.VMEM((2 * NBUF, CH * BW, D), jnp.float32),
            pltpu.VMEM((BW, D), jnp.float32),
            pltpu.VMEM((CH * BW,), jnp.int32),
            pltpu.VMEM((L,), jnp.float32),
            pltpu.VMEM_SHARED((CH * NS * BW, D), jnp.float32),
            pltpu.SemaphoreType.DMA((2 * NBUF,)),
            pltpu.SemaphoreType.DMA((2 * NBUF,)),
            pltpu.SemaphoreType.DMA,
        ],
    )
    def sc_pool(text_hbm, p_hbm, b_hbm, out_hbm,
                idx_v, rows_v, stage_v, scat_v, bias_v, acc_sh, gsems, ssems,
                stage_sem):
        cid = lax.axis_index("c")
        sid = lax.axis_index("s")
        wid = sid * NC + cid
        base = wid * BW
        sbase = sid * BW
        NSTEP = S // CH
        NSLOT = 2 * NBUF
        ROUNDS = NSTEP // NSLOT
        assert S % CH == 0 and NSTEP % NSLOT == 0

        def stage_row(r, _):
            s = r // CH
            c = r % CH
            pltpu.async_copy(text_hbm.at[pl.ds(r * B + base, BW)],
                             idx_v.at[s, pl.ds(c * BW, BW)], stage_sem)
            return 0
        lax.fori_loop(0, S, stage_row, 0)
        pltpu.sync_copy(b_hbm, bias_v)
        bias = bias_v[...]

        def stage_drain(r, _):
            pltpu.make_async_copy(
                text_hbm.at[pl.ds(r * B + base, BW)],
                idx_v.at[r // CH, pl.ds((r % CH) * BW, BW)], stage_sem).wait()
            return 0
        lax.fori_loop(0, S, stage_drain, 0)

        def init_row(j, _):
            stage_v[j] = bias
            return 0
        lax.fori_loop(0, BW, init_row, 0)
        pltpu.sync_copy(stage_v, acc_sh.at[pl.ds(sbase, BW)])

        def zero_row(j, _):
            stage_v[j] = jnp.zeros((L,), jnp.float32)
            return 0
        lax.fori_loop(0, BW, zero_row, 0)
        for c in range(1, CH):
            pltpu.sync_copy(stage_v, acc_sh.at[pl.ds(c * NS * BW + sbase, BW)])

        PERB = BW // L
        def init_scat(i, _):
            c = i // PERB
            scat_v[pl.ds(i * L, L)] = (
                lax.iota(jnp.int32, L)
                + (c * (NS * BW) + sbase + (i % PERB) * L))
            return 0
        lax.fori_loop(0, CH * BW // L, init_scat, 0)

        def gather_start(s, slot):
            pltpu.async_copy(p_hbm.at[idx_v.at[s]], rows_v.at[slot],
                             gsems.at[slot])

        def gather_wait(s, slot):
            pltpu.make_async_copy(
                p_hbm.at[idx_v.at[s]], rows_v.at[slot], gsems.at[slot]).wait()

        def scat_start(slot):
            pltpu.async_copy(rows_v.at[slot], acc_sh.at[scat_v],
                             ssems.at[slot], add=True)

        def scat_wait(slot):
            pltpu.make_async_copy(rows_v.at[slot], acc_sh.at[scat_v],
                                  ssems.at[slot]).wait()

        for b in range(NBUF):
            gather_start(b, b)

        for b in range(NSLOT):
            gather_wait(b, b)
            scat_start(b)
            if b + NBUF < NSLOT:
                gather_start(b + NBUF, (b + NBUF) % NSLOT)
            else:
                scat_wait((b + NBUF) % NSLOT)
                gather_start(b + NBUF, (b + NBUF) % NSLOT)

        def steady(k, _):
            s0 = k * NSLOT
            for b in range(NSLOT):
                s = s0 + b
                gather_wait(s, b)
                scat_start(b)
                slot2 = (b + NBUF) % NSLOT
                scat_wait(slot2)
                gather_start(s + NBUF, slot2)
            return 0
        lax.fori_loop(1, ROUNDS - 1, steady, 0)

        s0 = NSTEP - NSLOT
        for b in range(NSLOT):
            s = s0 + b
            gather_wait(s, b)
            scat_start(b)
            if s + NBUF < NSTEP:
                slot2 = (b + NBUF) % NSLOT
                scat_wait(slot2)
                gather_start(s + NBUF, slot2)

        for b in range(NSLOT):
            scat_wait(b)

        for c in range(CH):
            pltpu.sync_copy(acc_sh.at[pl.ds(c * NS * BW + sbase, BW)],
                            rows_v.at[0, pl.ds(c * BW, BW)])

        def sum_banks(j, _):
            acc = rows_v[0, j]
            for c in range(1, CH):
                acc = acc + rows_v[0, c * BW + j]
            stage_v[j] = acc
            return 0
        lax.fori_loop(0, BW, sum_banks, 0)
        pltpu.sync_copy(stage_v, out_hbm.at[pl.ds(base, BW)])

    return sc_pool


def kernel(text, emb_table, fc_w, fc_b):
    S, B = text.shape
    V = emb_table.shape[0]
    D = fc_w.shape[0]
    BLKC, PK = 4096, 128 // D
    SUB = BLKC // PK
    g = jnp.minimum(text.reshape(S * B), jnp.int32(V - 1))
    gidx = ((g // BLKC) * SUB + (g % SUB)) * PK + (g % BLKC) // SUB
    p = _project(emb_table, fc_w, 1.0 / S)
    return _make_sc_pool(S, B, D)(gidx, p, fc_b)
